# baseline (speedup 1.0000x reference)
import functools

import jax
import jax.numpy as jnp
from jax import lax
from jax.experimental import pallas as pl
from jax.experimental.pallas import tpu as pltpu
from jax.experimental.pallas import tpu_sc as plsc

BATCH = 1024
SEQ = 200
DIM = 64
HALF = SEQ // 2
LANES = 16
NUM_CORES = 2
NUM_SUBCORES = 16
NW = NUM_CORES * NUM_SUBCORES
ROWS_PER_W = BATCH // NW


def _body(idx_hbm, tok_hbm, pos_hbm, out_hbm, idx_v, rows_v, pos_v, sem):
    wid = lax.axis_index("s") * NUM_CORES + lax.axis_index("c")
    base = wid * ROWS_PER_W

    pltpu.sync_copy(pos_hbm, pos_v)

    def row_body(r, carry):
        row = base + r
        pltpu.sync_copy(idx_hbm.at[row], idx_v)
        cp0 = pltpu.async_copy(tok_hbm.at[idx_v.at[0]],
                               rows_v.at[pl.ds(0, HALF)], sem)
        cp1 = pltpu.async_copy(tok_hbm.at[idx_v.at[1]],
                               rows_v.at[pl.ds(HALF, HALF)], sem)
        cp0.wait()
        cp1.wait()

        def add_body(rr, c2):
            for c in range(DIM // LANES):
                v = pos_v[rr, pl.ds(c * LANES, LANES)]
                plsc.addupdate(rows_v.at[rr, pl.ds(c * LANES, LANES)], v)
            return c2

        lax.fori_loop(0, SEQ, add_body, 0, unroll=2)
        pltpu.sync_copy(rows_v, out_hbm.at[row])
        return carry

    lax.fori_loop(0, ROWS_PER_W, row_body, 0)


@functools.partial(jax.jit, static_argnames=())
def _run(idx, token_table, position_table):
    mesh = plsc.VectorSubcoreMesh(core_axis_name="c", subcore_axis_name="s")
    f = functools.partial(
        pl.kernel,
        out_type=jax.ShapeDtypeStruct((BATCH, SEQ, DIM), jnp.float32),
        mesh=mesh,
        scratch_types=[
            pltpu.VMEM((2, HALF), jnp.int32),
            pltpu.VMEM((SEQ, DIM), jnp.float32),
            pltpu.VMEM((SEQ, DIM), jnp.float32),
            pltpu.SemaphoreType.DMA,
        ],
        compiler_params=pltpu.CompilerParams(use_tc_tiling_on_sc=False),
    )(_body)
    return f(idx, token_table, position_table)


def kernel(inputs, token_table, position_table):
    idx = inputs.reshape(BATCH, 2, HALF).astype(jnp.int32)
    return _run(idx, token_table, position_table)


# speedup vs baseline: 1.0653x; 1.0653x over previous
import functools

import jax
import jax.numpy as jnp
from jax import lax
from jax.experimental import pallas as pl
from jax.experimental.pallas import tpu as pltpu
from jax.experimental.pallas import tpu_sc as plsc

BATCH = 1024
SEQ = 200
DIM = 64
HALF = SEQ // 2
LANES = 16
NUM_CORES = 2
NUM_SUBCORES = 16
NW = NUM_CORES * NUM_SUBCORES
ROWS_PER_W = BATCH // NW
CHUNK = 2
NCHUNK = ROWS_PER_W // CHUNK


def _body(idx_hbm, tok_hbm, pos_hbm, out_hbm,
          idx_v, pos_v, buf0, buf1, gsem0, gsem1, wsem0, wsem1):
    wid = lax.axis_index("s") * NUM_CORES + lax.axis_index("c")
    base = wid * ROWS_PER_W

    pltpu.sync_copy(pos_hbm, pos_v)
    pltpu.sync_copy(idx_hbm.at[pl.ds(base, ROWS_PER_W)], idx_v)

    bufs = (buf0, buf1)
    gsems = (gsem0, gsem1)
    wsems = (wsem0, wsem1)

    def fire_gather(chunk, b):
        cps = []
        for j in range(CHUNK):
            rl = chunk * CHUNK + j
            for h in range(2):
                cps.append(pltpu.async_copy(
                    tok_hbm.at[idx_v.at[rl, h]],
                    bufs[b].at[j, pl.ds(h * HALF, HALF)],
                    gsems[b]))
        return cps

    gcps = [None, None]
    wcps = [None, None]
    gcps[0] = fire_gather(0, 0)

    for chunk in range(NCHUNK):
        b = chunk % 2
        nb = (chunk + 1) % 2
        if chunk + 1 < NCHUNK:
            if wcps[nb] is not None:
                wcps[nb].wait()
                wcps[nb] = None
            gcps[nb] = fire_gather(chunk + 1, nb)
        for cp in gcps[b]:
            cp.wait()

        for j in range(CHUNK):
            def add_body(r, c2, _j=j, _b=b):
                for c in range(DIM // LANES):
                    v = pos_v[r, pl.ds(c * LANES, LANES)]
                    plsc.addupdate(
                        bufs[_b].at[_j, r, pl.ds(c * LANES, LANES)], v)
                return c2
            lax.fori_loop(0, SEQ, add_body, 0, unroll=2)

        wcps[b] = pltpu.async_copy(
            bufs[b], out_hbm.at[pl.ds(base + chunk * CHUNK, CHUNK)], wsems[b])

    for b in range(2):
        if wcps[b] is not None:
            wcps[b].wait()


@jax.jit
def _run(idx, token_table, position_table):
    mesh = plsc.VectorSubcoreMesh(core_axis_name="c", subcore_axis_name="s")
    f = functools.partial(
        pl.kernel,
        out_type=jax.ShapeDtypeStruct((BATCH, SEQ, DIM), jnp.float32),
        mesh=mesh,
        scratch_types=[
            pltpu.VMEM((ROWS_PER_W, 2, HALF), jnp.int32),
            pltpu.VMEM((SEQ, DIM), jnp.float32),
            pltpu.VMEM((CHUNK, SEQ, DIM), jnp.float32),
            pltpu.VMEM((CHUNK, SEQ, DIM), jnp.float32),
            pltpu.SemaphoreType.DMA,
            pltpu.SemaphoreType.DMA,
            pltpu.SemaphoreType.DMA,
            pltpu.SemaphoreType.DMA,
        ],
        compiler_params=pltpu.CompilerParams(use_tc_tiling_on_sc=False),
    )(_body)
    return f(idx, token_table, position_table)


def kernel(inputs, token_table, position_table):
    idx = inputs.reshape(BATCH, 2, HALF).astype(jnp.int32)
    return _run(idx, token_table, position_table)
